# baseline (speedup 1.0000x reference)
import functools

import jax
import jax.numpy as jnp
from jax import lax
from jax.experimental import pallas as pl
from jax.experimental.pallas import tpu as pltpu
from jax.experimental.pallas import tpu_sc as plsc

_PAD = 8
_D = 300
_DPAD = 320
_NW = 32
_CHUNK = 128


def _sc_gather(tbl, idx, n_tok):
  per_w = n_tok // _NW
  n_chunk = per_w // _CHUNK
  mesh = plsc.VectorSubcoreMesh(core_axis_name="c", subcore_axis_name="s")

  @functools.partial(
      pl.kernel,
      out_type=jax.ShapeDtypeStruct((n_tok, _D), jnp.float32),
      mesh=mesh,
      scratch_types=[
          pltpu.VMEM((n_chunk, _CHUNK), jnp.int32),
          pltpu.VMEM((_CHUNK, _D), jnp.float32),
          pltpu.VMEM((_CHUNK, _D), jnp.float32),
          pltpu.SemaphoreType.DMA,
          pltpu.SemaphoreType.DMA,
          pltpu.SemaphoreType.DMA,
          pltpu.SemaphoreType.DMA,
      ],
      compiler_params=pltpu.CompilerParams(use_tc_tiling_on_sc=False),
  )
  def k(tbl_hbm, idx_hbm, out_hbm, idx_v, buf0, buf1, g0, g1, w0, w1):
    wid = lax.axis_index("s") * 2 + lax.axis_index("c")
    base = wid * per_w
    pltpu.sync_copy(idx_hbm.at[wid], idx_v)

    def gather(j, buf, sem):
      return pltpu.async_copy(tbl_hbm.at[idx_v.at[j]], buf, sem)

    def write(j, buf, sem):
      return pltpu.async_copy(
          buf, out_hbm.at[pl.ds(base + j * _CHUNK, _CHUNK)], sem
      )

    def body(t, carry):
      hg0 = gather(2 * t, buf0, g0)
      hg0.wait()
      hw0 = write(2 * t, buf0, w0)
      hg1 = gather(2 * t + 1, buf1, g1)
      hg1.wait()
      hw1 = write(2 * t + 1, buf1, w1)
      hw0.wait()
      hw1.wait()
      return carry

    lax.fori_loop(0, n_chunk // 2, body, 0)

  return k(tbl, idx)


def kernel(inputs, emb_table):
  b, l = inputs.shape
  n_tok = b * l
  per_w = n_tok // _NW
  idx = inputs.reshape(_NW, per_w // _CHUNK, _CHUNK).astype(jnp.int32)
  tbl = emb_table.at[0].set(0.0).at[_PAD].set(0.0)
  out = _sc_gather(tbl, idx, n_tok)
  return out.reshape(b, l, _D)


# speedup vs baseline: 1.7702x; 1.7702x over previous
import functools

import jax
import jax.numpy as jnp
from jax import lax
from jax.experimental import pallas as pl
from jax.experimental.pallas import tpu as pltpu
from jax.experimental.pallas import tpu_sc as plsc

_PAD = 8
_D = 300
_Q = 4
_QD = _D * _Q
_QDPAD = _QD + 16
_NV = 9
_NW = 32
_CHUNK = 32


def _sc_gather(qtbl, qidx, n_quad):
  per_w = n_quad // _NW
  n_chunk = per_w // _CHUNK
  mesh = plsc.VectorSubcoreMesh(core_axis_name="c", subcore_axis_name="s")

  @functools.partial(
      pl.kernel,
      out_type=jax.ShapeDtypeStruct((n_quad, _QD), jnp.float32),
      mesh=mesh,
      scratch_types=[
          pltpu.VMEM((n_chunk, _CHUNK), jnp.int32),
          pltpu.VMEM((_CHUNK, _QDPAD), jnp.float32),
          pltpu.VMEM((_CHUNK, _QDPAD), jnp.float32),
          pltpu.SemaphoreType.DMA,
          pltpu.SemaphoreType.DMA,
          pltpu.SemaphoreType.DMA,
          pltpu.SemaphoreType.DMA,
      ],
      compiler_params=pltpu.CompilerParams(use_tc_tiling_on_sc=False),
  )
  def k(tbl_hbm, idx_hbm, out_hbm, idx_v, buf0, buf1, g0, g1, w0, w1):
    wid = lax.axis_index("s") * 2 + lax.axis_index("c")
    base = wid * per_w
    pltpu.sync_copy(idx_hbm.at[wid], idx_v)

    def gather(j, buf, sem):
      return pltpu.async_copy(tbl_hbm.at[idx_v.at[j]], buf, sem)

    def write(j, buf, sem):
      return pltpu.async_copy(
          buf.at[:, pl.ds(0, _QD)],
          out_hbm.at[pl.ds(base + j * _CHUNK, _CHUNK)],
          sem,
      )

    def body(t, carry):
      hg0 = gather(2 * t, buf0, g0)
      hg0.wait()
      hw0 = write(2 * t, buf0, w0)
      hg1 = gather(2 * t + 1, buf1, g1)
      hg1.wait()
      hw1 = write(2 * t + 1, buf1, w1)
      hw0.wait()
      hw1.wait()
      return carry

    lax.fori_loop(0, n_chunk // 2, body, 0)

  return k(qtbl, qidx)


def kernel(inputs, emb_table):
  b, l = inputs.shape
  n_tok = b * l
  n_quad = n_tok // _Q
  per_w = n_quad // _NW

  tbl = emb_table.at[0].set(0.0).at[_PAD].set(0.0)
  n = _NV
  qt = jnp.concatenate(
      [
          jnp.broadcast_to(tbl[:, None, None, None, :], (n, n, n, n, _D)),
          jnp.broadcast_to(tbl[None, :, None, None, :], (n, n, n, n, _D)),
          jnp.broadcast_to(tbl[None, None, :, None, :], (n, n, n, n, _D)),
          jnp.broadcast_to(tbl[None, None, None, :, :], (n, n, n, n, _D)),
      ],
      axis=-1,
  ).reshape(n * n * n * n, _QD)
  qt = jnp.pad(qt, ((0, 0), (0, _QDPAD - _QD)))

  iq = inputs.reshape(-1, _Q).astype(jnp.int32)
  qidx = ((iq[:, 0] * n + iq[:, 1]) * n + iq[:, 2]) * n + iq[:, 3]
  qidx = qidx.reshape(_NW, per_w // _CHUNK, _CHUNK)

  out = _sc_gather(qt, qidx, n_quad)
  return out.reshape(b, l, _D)


# speedup vs baseline: 1.7741x; 1.0022x over previous
import functools

import jax
import jax.numpy as jnp
from jax import lax
from jax.experimental import pallas as pl
from jax.experimental.pallas import tpu as pltpu
from jax.experimental.pallas import tpu_sc as plsc

_PAD = 8
_D = 300
_Q = 4
_QD = _D * _Q
_NV = 9
_NW = 32
_CHUNK = 32


def _sc_gather(qtbl, qidx, n_quad):
  per_w = n_quad // _NW
  n_chunk = per_w // _CHUNK
  mesh = plsc.VectorSubcoreMesh(core_axis_name="c", subcore_axis_name="s")

  @functools.partial(
      pl.kernel,
      out_type=jax.ShapeDtypeStruct((n_quad, _QD), jnp.float32),
      mesh=mesh,
      scratch_types=[
          pltpu.VMEM((n_chunk, _CHUNK), jnp.int32),
          pltpu.VMEM((_CHUNK, _QD), jnp.float32),
          pltpu.VMEM((_CHUNK, _QD), jnp.float32),
          pltpu.SemaphoreType.DMA,
          pltpu.SemaphoreType.DMA,
          pltpu.SemaphoreType.DMA,
          pltpu.SemaphoreType.DMA,
      ],
      compiler_params=pltpu.CompilerParams(use_tc_tiling_on_sc=False),
  )
  def k(tbl_hbm, idx_hbm, out_hbm, idx_v, buf0, buf1, g0, g1, w0, w1):
    wid = lax.axis_index("s") * 2 + lax.axis_index("c")
    base = wid * per_w
    pltpu.sync_copy(idx_hbm.at[wid], idx_v)

    def gather(j, buf, sem):
      return pltpu.async_copy(tbl_hbm.at[idx_v.at[j]], buf, sem)

    def write(j, buf, sem):
      return pltpu.async_copy(
          buf, out_hbm.at[pl.ds(base + j * _CHUNK, _CHUNK)], sem
      )

    def body(t, carry):
      hg0 = gather(2 * t, buf0, g0)
      hg0.wait()
      hw0 = write(2 * t, buf0, w0)
      hg1 = gather(2 * t + 1, buf1, g1)
      hg1.wait()
      hw1 = write(2 * t + 1, buf1, w1)
      hw0.wait()
      hw1.wait()
      return carry

    lax.fori_loop(0, n_chunk // 2, body, 0)

  return k(qtbl, qidx)


def kernel(inputs, emb_table):
  b, l = inputs.shape
  n_tok = b * l
  n_quad = n_tok // _Q
  per_w = n_quad // _NW

  tbl = emb_table.at[0].set(0.0).at[_PAD].set(0.0)
  n = _NV
  span = jnp.arange(n, dtype=jnp.int32)
  midx = jnp.stack(
      jnp.meshgrid(span, span, span, span, indexing="ij"), axis=-1
  ).reshape(-1)
  qt = jnp.take(tbl, midx, axis=0).reshape(n * n * n * n, _QD)

  iq = inputs.reshape(-1, _Q).astype(jnp.int32)
  qidx = ((iq[:, 0] * n + iq[:, 1]) * n + iq[:, 2]) * n + iq[:, 3]
  qidx = qidx.reshape(_NW, per_w // _CHUNK, _CHUNK)

  out = _sc_gather(qt, qidx, n_quad)
  return out.reshape(b, l, _D)


# speedup vs baseline: 1.8231x; 1.0277x over previous
import functools

import jax
import jax.numpy as jnp
from jax import lax
from jax.experimental import pallas as pl
from jax.experimental.pallas import tpu as pltpu
from jax.experimental.pallas import tpu_sc as plsc

_PAD = 8
_D = 300
_Q = 4
_QD = _D * _Q
_NV = 9
_NW = 32
_CHUNK = 32


def _sc_gather(qtbl, qidx, n_quad):
  per_w = n_quad // _NW
  n_chunk = per_w // _CHUNK
  mesh = plsc.VectorSubcoreMesh(core_axis_name="c", subcore_axis_name="s")

  @functools.partial(
      pl.kernel,
      out_type=jax.ShapeDtypeStruct((n_quad, _QD), jnp.float32),
      mesh=mesh,
      scratch_types=[
          pltpu.VMEM((n_chunk, _CHUNK), jnp.int32),
          pltpu.VMEM((_CHUNK, _QD), jnp.float32),
          pltpu.VMEM((_CHUNK, _QD), jnp.float32),
          pltpu.SemaphoreType.DMA,
          pltpu.SemaphoreType.DMA,
          pltpu.SemaphoreType.DMA,
          pltpu.SemaphoreType.DMA,
      ],
      compiler_params=pltpu.CompilerParams(use_tc_tiling_on_sc=False),
  )
  def k(tbl_hbm, idx_hbm, out_hbm, idx_v, buf0, buf1, g0, g1, w0, w1):
    wid = lax.axis_index("s") * 2 + lax.axis_index("c")
    base = wid * per_w
    pltpu.sync_copy(idx_hbm.at[wid], idx_v)

    def gather(j, buf, sem):
      return pltpu.async_copy(tbl_hbm.at[idx_v.at[j]], buf, sem)

    def write(j, buf, sem):
      return pltpu.async_copy(
          buf, out_hbm.at[pl.ds(base + j * _CHUNK, _CHUNK)], sem
      )

    def body(t, carry):
      hg0 = gather(2 * t, buf0, g0)
      hg0.wait()
      hw0 = write(2 * t, buf0, w0)
      hg1 = gather(2 * t + 1, buf1, g1)
      hg1.wait()
      hw1 = write(2 * t + 1, buf1, w1)
      hw0.wait()
      hw1.wait()
      return carry

    lax.fori_loop(0, n_chunk // 2, body, 0)

  return k(qtbl, qidx)


def kernel(inputs, emb_table):
  b, l = inputs.shape
  n_tok = b * l
  n_quad = n_tok // _Q
  per_w = n_quad // _NW

  tbl = emb_table.at[0].set(0.0).at[_PAD].set(0.0)
  n = _NV
  span = jnp.arange(n * n * n * n, dtype=jnp.int32)
  qt = jnp.concatenate(
      [
          jnp.take(tbl, (span // (n * n * n)) % n, axis=0),
          jnp.take(tbl, (span // (n * n)) % n, axis=0),
          jnp.take(tbl, (span // n) % n, axis=0),
          jnp.take(tbl, span % n, axis=0),
      ],
      axis=1,
  )

  iq = inputs.reshape(-1, _Q).astype(jnp.int32)
  qidx = ((iq[:, 0] * n + iq[:, 1]) * n + iq[:, 2]) * n + iq[:, 3]
  qidx = qidx.reshape(_NW, per_w // _CHUNK, _CHUNK)

  out = _sc_gather(qt, qidx, n_quad)
  return out.reshape(b, l, _D)
